# baseline (speedup 1.0000x reference)
import functools

import jax
import jax.numpy as jnp
from jax import lax
from jax.experimental import pallas as pl
from jax.experimental.pallas import tpu as pltpu
from jax.experimental.pallas import tpu_sc as plsc

_IDX_MINOR = 128
_IPC = 2


def _sc_gather(idx2d, table):
    n_rows, minor = idx2d.shape
    assert minor == _IDX_MINOR
    vocab, d = table.shape
    info = plsc.get_sparse_core_info()
    nc, ns = info.num_cores, info.num_subcores
    nw = nc * ns
    rows_per_w = n_rows // nw
    assert rows_per_w * nw == n_rows
    n_ch = rows_per_w // _IPC
    assert n_ch * _IPC == rows_per_w and n_ch >= 6
    ch = _IPC * _IDX_MINOR

    mesh = plsc.VectorSubcoreMesh(core_axis_name="c", subcore_axis_name="s")

    @functools.partial(
        pl.kernel,
        mesh=mesh,
        compiler_params=pltpu.CompilerParams(use_tc_tiling_on_sc=True),
        out_type=jax.ShapeDtypeStruct((n_rows * _IDX_MINOR, d), jnp.float32),
        scratch_types=[
            pltpu.VMEM((2, _IPC, _IDX_MINOR), jnp.int32),
            pltpu.VMEM((2, ch, d), jnp.float32),
            pltpu.SemaphoreType.DMA((2,)),
            pltpu.SemaphoreType.DMA,
            pltpu.SemaphoreType.DMA((2,)),
        ],
    )
    def k(idx_hbm, table_hbm, out_hbm, idx_v, rows_v, isem, gsem, ssem):
        wid = lax.axis_index("s") * nc + lax.axis_index("c")
        irow0 = wid * rows_per_w

        def idx_start(g, b):
            pltpu.async_copy(
                idx_hbm.at[pl.ds(irow0 + g * _IPC, _IPC)], idx_v.at[b],
                isem.at[b])

        def idx_wait(b):
            pltpu.make_async_copy(
                idx_hbm.at[pl.ds(0, _IPC)], idx_v.at[b], isem.at[b]).wait()

        def gather(b):
            for gi in range(_IDX_MINOR // 16):
                for j in range(_IPC):
                    v = idx_v[b, j, pl.ds(gi * 16, 16)]
                    for l in range(16):
                        r = v[l]
                        pltpu.async_copy(
                            table_hbm.at[pl.ds(r, 1)],
                            rows_v.at[
                                b, pl.ds(j * _IDX_MINOR + gi * 16 + l, 1)],
                            gsem)

            pltpu.make_async_copy(
                table_hbm.at[pl.ds(0, ch)], rows_v.at[b], gsem).wait()

        def store_start(g, b):
            pltpu.async_copy(
                rows_v.at[b],
                out_hbm.at[pl.ds((irow0 + g * _IPC) * _IDX_MINOR, ch)],
                ssem.at[b])

        def store_wait(b):
            pltpu.make_async_copy(
                rows_v.at[b], out_hbm.at[pl.ds(0, ch)], ssem.at[b]).wait()

        idx_start(0, 0)
        idx_start(1, 1)
        for b in (0, 1):
            idx_wait(b)
            gather(b)
            idx_start(b + 2, b)
            store_start(b, b)

        def pair(p, _):
            for b in (0, 1):
                g = 2 * p + b
                idx_wait(b)
                store_wait(b)
                gather(b)
                idx_start(g + 2, b)
                store_start(g, b)
            return _

        lax.fori_loop(1, n_ch // 2 - 1, pair, None)

        for b in (0, 1):
            g = n_ch - 2 + b
            idx_wait(b)
            store_wait(b)
            gather(b)
            store_start(g, b)
        for b in (0, 1):
            store_wait(b)

    return k(idx2d, table)


def _pe_body(o_ref):
    h, d = o_ref.shape
    pos = lax.broadcasted_iota(jnp.int32, (h, d), 0).astype(jnp.float32)
    col = lax.broadcasted_iota(jnp.int32, (h, d), 1)
    half = jnp.floor(col.astype(jnp.float32) * 0.5)
    ln10000 = 9.210340371976184
    inv_freq = jnp.exp(half * (-2.0 * ln10000 / d))
    ang = pos * inv_freq
    odd = (col % 2) == 1
    o_ref[...] = jnp.where(odd, jnp.cos(ang), jnp.sin(ang))


def kernel(inputs, table):
    batch, hist = inputs.shape
    d = table.shape[1]
    idx2d = inputs.astype(jnp.int32).reshape(-1, _IDX_MINOR)
    flat = _sc_gather(idx2d, table)
    pe = pl.pallas_call(
        _pe_body,
        out_shape=jax.ShapeDtypeStruct((hist, d), jnp.float32),
    )()
    return flat.reshape(batch, hist, d), pe


# speedup vs baseline: 1.0309x; 1.0309x over previous
import functools

import jax
import jax.numpy as jnp
from jax import lax
from jax.experimental import pallas as pl
from jax.experimental.pallas import tpu as pltpu
from jax.experimental.pallas import tpu_sc as plsc

_IDX_MINOR = 128
_IPC = 2


def _sc_gather(idx2d, table):
    n_rows, minor = idx2d.shape
    assert minor == _IDX_MINOR
    vocab, d = table.shape
    info = plsc.get_sparse_core_info()
    nc, ns = info.num_cores, info.num_subcores
    nw = nc * ns
    rows_per_w = n_rows // nw
    assert rows_per_w * nw == n_rows
    n_ch = rows_per_w // _IPC
    assert n_ch * _IPC == rows_per_w and n_ch >= 6
    ch = _IPC * _IDX_MINOR

    mesh = plsc.VectorSubcoreMesh(core_axis_name="c", subcore_axis_name="s")

    @functools.partial(
        pl.kernel,
        mesh=mesh,
        compiler_params=pltpu.CompilerParams(use_tc_tiling_on_sc=True),
        out_type=jax.ShapeDtypeStruct((n_rows * _IDX_MINOR, d), jnp.float32),
        scratch_types=[
            pltpu.VMEM((2, _IPC, _IDX_MINOR), jnp.int32),
            pltpu.VMEM((2, ch, d), jnp.float32),
            pltpu.SemaphoreType.DMA((2,)),
            pltpu.SemaphoreType.DMA,
            pltpu.SemaphoreType.DMA((2,)),
        ],
    )
    def k(idx_hbm, table_hbm, out_hbm, idx_v, rows_v, isem, gsem, ssem):
        wid = lax.axis_index("s") * nc + lax.axis_index("c")
        irow0 = wid * rows_per_w

        def idx_start(g, b):
            pltpu.async_copy(
                idx_hbm.at[pl.ds(irow0 + g * _IPC, _IPC)], idx_v.at[b],
                isem.at[b])

        def idx_wait(b):
            pltpu.make_async_copy(
                idx_hbm.at[pl.ds(0, _IPC)], idx_v.at[b], isem.at[b]).wait()

        def gather(b):
            def issue(gi, _):
                for j in range(_IPC):
                    v = idx_v[b, j, pl.ds(gi * 16, 16)]
                    for l in range(16):
                        r = v[l]
                        pltpu.async_copy(
                            table_hbm.at[pl.ds(r, 1)],
                            rows_v.at[
                                b, pl.ds(j * _IDX_MINOR + gi * 16 + l, 1)],
                            gsem)
                return _
            lax.fori_loop(0, _IDX_MINOR // 16, issue, None)

            pltpu.make_async_copy(
                table_hbm.at[pl.ds(0, ch)], rows_v.at[b], gsem).wait()

        def store_start(g, b):
            pltpu.async_copy(
                rows_v.at[b],
                out_hbm.at[pl.ds((irow0 + g * _IPC) * _IDX_MINOR, ch)],
                ssem.at[b])

        def store_wait(b):
            pltpu.make_async_copy(
                rows_v.at[b], out_hbm.at[pl.ds(0, ch)], ssem.at[b]).wait()

        idx_start(0, 0)
        idx_start(1, 1)
        for b in (0, 1):
            idx_wait(b)
            gather(b)
            idx_start(b + 2, b)
            store_start(b, b)

        def pair(p, _):
            for b in (0, 1):
                g = 2 * p + b
                idx_wait(b)
                store_wait(b)
                gather(b)
                idx_start(g + 2, b)
                store_start(g, b)
            return _

        lax.fori_loop(1, n_ch // 2 - 1, pair, None)

        for b in (0, 1):
            g = n_ch - 2 + b
            idx_wait(b)
            store_wait(b)
            gather(b)
            store_start(g, b)
        for b in (0, 1):
            store_wait(b)

    return k(idx2d, table)


def _pe_body(o_ref):
    h, d = o_ref.shape
    pos = lax.broadcasted_iota(jnp.int32, (h, d), 0).astype(jnp.float32)
    col = lax.broadcasted_iota(jnp.int32, (h, d), 1)
    half = jnp.floor(col.astype(jnp.float32) * 0.5)
    ln10000 = 9.210340371976184
    inv_freq = jnp.exp(half * (-2.0 * ln10000 / d))
    ang = pos * inv_freq
    odd = (col % 2) == 1
    o_ref[...] = jnp.where(odd, jnp.cos(ang), jnp.sin(ang))


def kernel(inputs, table):
    batch, hist = inputs.shape
    d = table.shape[1]
    idx2d = inputs.astype(jnp.int32).reshape(-1, _IDX_MINOR)
    flat = _sc_gather(idx2d, table)
    pe = pl.pallas_call(
        _pe_body,
        out_shape=jax.ShapeDtypeStruct((hist, d), jnp.float32),
    )()
    return flat.reshape(batch, hist, d), pe


# speedup vs baseline: 1.0345x; 1.0035x over previous
import functools

import jax
import jax.numpy as jnp
from jax import lax
from jax.experimental import pallas as pl
from jax.experimental.pallas import tpu as pltpu
from jax.experimental.pallas import tpu_sc as plsc

_IDX_MINOR = 128
_IPC = 2


def _sc_gather(idx2d, table):
    n_rows, minor = idx2d.shape
    assert minor == _IDX_MINOR
    vocab, d = table.shape
    info = plsc.get_sparse_core_info()
    nc, ns = info.num_cores, info.num_subcores
    nw = nc * ns
    rows_per_w = n_rows // nw
    assert rows_per_w * nw == n_rows
    n_ch = rows_per_w // _IPC
    assert n_ch * _IPC == rows_per_w and n_ch >= 6
    ch = _IPC * _IDX_MINOR

    mesh = plsc.VectorSubcoreMesh(core_axis_name="c", subcore_axis_name="s")

    @functools.partial(
        pl.kernel,
        mesh=mesh,
        compiler_params=pltpu.CompilerParams(use_tc_tiling_on_sc=True),
        out_type=jax.ShapeDtypeStruct((n_rows * _IDX_MINOR, d), jnp.float32),
        scratch_types=[
            pltpu.VMEM((2, _IPC, _IDX_MINOR), jnp.int32),
            pltpu.VMEM((2, ch, d), jnp.float32),
            pltpu.SemaphoreType.DMA((2,)),
            pltpu.SemaphoreType.DMA((2,)),
            pltpu.SemaphoreType.DMA((2,)),
        ],
    )
    def k(idx_hbm, table_hbm, out_hbm, idx_v, rows_v, isem, gsem, ssem):
        wid = lax.axis_index("s") * nc + lax.axis_index("c")
        irow0 = wid * rows_per_w

        def idx_start(g, b):
            pltpu.async_copy(
                idx_hbm.at[pl.ds(irow0 + g * _IPC, _IPC)], idx_v.at[b],
                isem.at[b])

        def idx_wait(b):
            pltpu.make_async_copy(
                idx_hbm.at[pl.ds(0, _IPC)], idx_v.at[b], isem.at[b]).wait()

        def issue_chunk(b):
            def issue(gi, _):
                for j in range(_IPC):
                    v = idx_v[b, j, pl.ds(gi * 16, 16)]
                    for l in range(16):
                        r = v[l]
                        pltpu.async_copy(
                            table_hbm.at[pl.ds(r, 1)],
                            rows_v.at[
                                b, pl.ds(j * _IDX_MINOR + gi * 16 + l, 1)],
                            gsem.at[b])
                return _
            lax.fori_loop(0, _IDX_MINOR // 16, issue, None)

        def drain_chunk(b):
            pltpu.make_async_copy(
                table_hbm.at[pl.ds(0, ch)], rows_v.at[b], gsem.at[b]).wait()

        def store_start(g, b):
            pltpu.async_copy(
                rows_v.at[b],
                out_hbm.at[pl.ds((irow0 + g * _IPC) * _IDX_MINOR, ch)],
                ssem.at[b])

        def store_wait(b):
            pltpu.make_async_copy(
                rows_v.at[b], out_hbm.at[pl.ds(0, ch)], ssem.at[b]).wait()

        idx_start(0, 0)
        idx_start(1, 1)
        idx_wait(0)
        issue_chunk(0)
        idx_start(2, 0)
        idx_wait(1)
        issue_chunk(1)
        drain_chunk(0)
        store_start(0, 0)
        idx_start(3, 1)

        def pair(p, _):
            for b in (0, 1):
                g = 2 * p + b
                idx_wait(b)
                store_wait(b)
                issue_chunk(b)
                drain_chunk(b ^ 1)
                store_start(g - 1, b ^ 1)
                idx_start(g + 2, b)
            return _

        lax.fori_loop(1, n_ch // 2 - 1, pair, None)

        for b in (0, 1):
            g = n_ch - 2 + b
            idx_wait(b)
            store_wait(b)
            issue_chunk(b)
            drain_chunk(b ^ 1)
            store_start(g - 1, b ^ 1)
        drain_chunk(1)
        store_start(n_ch - 1, 1)
        for b in (0, 1):
            store_wait(b)

    return k(idx2d, table)


def _pe_body(o_ref):
    h, d = o_ref.shape
    pos = lax.broadcasted_iota(jnp.int32, (h, d), 0).astype(jnp.float32)
    col = lax.broadcasted_iota(jnp.int32, (h, d), 1)
    half = jnp.floor(col.astype(jnp.float32) * 0.5)
    ln10000 = 9.210340371976184
    inv_freq = jnp.exp(half * (-2.0 * ln10000 / d))
    ang = pos * inv_freq
    odd = (col % 2) == 1
    o_ref[...] = jnp.where(odd, jnp.cos(ang), jnp.sin(ang))


def kernel(inputs, table):
    batch, hist = inputs.shape
    d = table.shape[1]
    idx2d = inputs.astype(jnp.int32).reshape(-1, _IDX_MINOR)
    flat = _sc_gather(idx2d, table)
    pe = pl.pallas_call(
        _pe_body,
        out_shape=jax.ShapeDtypeStruct((hist, d), jnp.float32),
    )()
    return flat.reshape(batch, hist, d), pe
